# baseline (speedup 1.0000x reference)
import functools

import jax
import jax.numpy as jnp
from jax import lax
from jax.experimental import pallas as pl
from jax.experimental.pallas import tpu as pltpu
from jax.experimental.pallas import tpu_sc as plsc

_B = 100
_NC, _NS = 2, 16
_NW = _NC * _NS
_CH = 128
_F32 = jnp.float32


def _sds(shape, dtype=_F32):
    return jax.ShapeDtypeStruct(shape, dtype)


def _rup(n, m):
    return ((n + m - 1) // m) * m


def _silu(x):
    return x / (1.0 + jnp.exp(-x))


def _tc_dense(x, w, b, br):
    n, k = x.shape
    ko = w.shape[1]

    def body(x_ref, w_ref, b_ref, o_ref):
        o_ref[...] = (jnp.dot(x_ref[...], w_ref[...],
                              preferred_element_type=_F32) + b_ref[...])

    return pl.pallas_call(
        body, grid=(n // br,),
        in_specs=[pl.BlockSpec((br, k), lambda i: (i, 0)),
                  pl.BlockSpec((k, ko), lambda i: (0, 0)),
                  pl.BlockSpec((1, ko), lambda i: (0, 0))],
        out_specs=pl.BlockSpec((br, ko), lambda i: (i, 0)),
        out_shape=_sds((n, ko)))(x, w, b.reshape(1, -1))


def _tc_mlp2(x, w1, b1, w2, b2, br):
    n, k = x.shape
    km = w1.shape[1]
    ko = w2.shape[1]

    def body(x_ref, w1_ref, b1_ref, w2_ref, b2_ref, o_ref):
        t = jnp.dot(x_ref[...], w1_ref[...], preferred_element_type=_F32) + b1_ref[...]
        o_ref[...] = (jnp.dot(_silu(t), w2_ref[...],
                              preferred_element_type=_F32) + b2_ref[...])

    return pl.pallas_call(
        body, grid=(n // br,),
        in_specs=[pl.BlockSpec((br, k), lambda i: (i, 0)),
                  pl.BlockSpec((k, km), lambda i: (0, 0)),
                  pl.BlockSpec((1, km), lambda i: (0, 0)),
                  pl.BlockSpec((km, ko), lambda i: (0, 0)),
                  pl.BlockSpec((1, ko), lambda i: (0, 0))],
        out_specs=pl.BlockSpec((br, ko), lambda i: (i, 0)),
        out_shape=_sds((n, ko)))(x, w1, b1.reshape(1, -1), w2, b2.reshape(1, -1))


def _tc_proj(x, wcat, br):
    n = x.shape[0]

    def body(x_ref, w_ref, o0_ref, o1_ref, o2_ref):
        y = jnp.dot(x_ref[...], w_ref[...], preferred_element_type=_F32)
        for t, o_ref in enumerate((o0_ref, o1_ref, o2_ref)):
            for c in range(2):
                o_ref[c] = y[:, t * 128 + c * 64: t * 128 + (c + 1) * 64]

    return pl.pallas_call(
        body, grid=(n // br,),
        in_specs=[pl.BlockSpec((br, 128), lambda i: (i, 0)),
                  pl.BlockSpec((128, 384), lambda i: (0, 0))],
        out_specs=[pl.BlockSpec((2, br, 64), lambda i: (0, i, 0))] * 3,
        out_shape=[_sds((2, n, 64))] * 3)(x, wcat)


def _tc_t1(a, b, br):
    n = a.shape[0]

    def body(a_ref, b_ref, o_ref):
        av = a_ref[...]
        bv = b_ref[...]
        c = 0.5 * (av + bv)
        d2 = jnp.sum((av - bv) ** 2, axis=1, keepdims=True)
        d = jnp.sqrt(d2 + 1e-12)
        lane = lax.broadcasted_iota(jnp.int32, (br, 16), 1)
        o_ref[...] = jnp.where(lane < 3, c, 0.0) + jnp.where(lane == 3, d, 0.0)

    return pl.pallas_call(
        body, grid=(n // br,),
        in_specs=[pl.BlockSpec((br, 16), lambda i: (i, 0)),
                  pl.BlockSpec((br, 16), lambda i: (i, 0))],
        out_specs=pl.BlockSpec((br, 16), lambda i: (i, 0)),
        out_shape=_sds((n, 16)))(a, b)


def _tc_stats(rr, ss, etrue, be):
    ep = rr.shape[0]

    def body(r_ref, s_ref, o_ref):
        i = pl.program_id(0)

        @pl.when(i == 0)
        def _():
            o_ref[...] = jnp.zeros_like(o_ref)

        a = r_ref[...]
        b = s_ref[...]
        lane = lax.broadcasted_iota(jnp.int32, (be, 16), 1)
        sub = lax.broadcasted_iota(jnp.int32, (be, 16), 0)
        valid = (sub + i * be) < etrue
        d = jnp.where((lane < 3) & valid, a - b, 0.0)
        dd = jnp.sum(d * d, axis=1, keepdims=True)
        dr = jnp.sum(jnp.where((lane == 3) & valid, a, 0.0), axis=1, keepdims=True)
        ds = jnp.sum(jnp.where((lane == 3) & valid, b, 0.0), axis=1, keepdims=True)
        vals = [jnp.sum(dd), jnp.sum(dr), jnp.sum(ds),
                jnp.sum(dd * dd), jnp.sum(dr * dr), jnp.sum(ds * ds)]
        l2 = lax.broadcasted_iota(jnp.int32, (8, 128), 1)
        s2 = lax.broadcasted_iota(jnp.int32, (8, 128), 0)
        p = jnp.zeros((8, 128), _F32)
        for k, v in enumerate(vals):
            p = p + jnp.where((s2 == 0) & (l2 == k), v, 0.0)
        o_ref[...] += p

    return pl.pallas_call(
        body, grid=(ep // be,),
        in_specs=[pl.BlockSpec((be, 16), lambda i: (i, 0)),
                  pl.BlockSpec((be, 16), lambda i: (i, 0))],
        out_specs=pl.BlockSpec((8, 128), lambda i: (0, 0)),
        out_shape=_sds((8, 128)))(rr, ss)


def _tc_invp(rr, ss, mflat, cvec, etrue, be):
    ep = rr.shape[0]

    def body(r_ref, s_ref, m_ref, c_ref, o_ref):
        i = pl.program_id(0)
        a = r_ref[...]
        b = s_ref[...]
        lane = lax.broadcasted_iota(jnp.int32, (be, 16), 1)
        d = jnp.where(lane < 3, a - b, 0.0)
        dd = jnp.sum(d * d, axis=1, keepdims=True)
        dr = a[:, 3:4]
        dsv = b[:, 3:4]
        sub = lax.broadcasted_iota(jnp.int32, (be, 1), 0)
        pad = (sub + i * be) >= etrue
        m = m_ref[...]
        cv = c_ref[...]
        for l in range(4):
            y = (dd * m[3 * l + 0:3 * l + 1, :] + dr * m[3 * l + 1:3 * l + 2, :]
                 + dsv * m[3 * l + 2:3 * l + 3, :] + cv[l:l + 1, :])
            o_ref[l] = jnp.where(pad, -1e4, y)

    return pl.pallas_call(
        body, grid=(ep // be,),
        in_specs=[pl.BlockSpec((be, 16), lambda i: (i, 0)),
                  pl.BlockSpec((be, 16), lambda i: (i, 0)),
                  pl.BlockSpec((12, 128), lambda i: (0, 0)),
                  pl.BlockSpec((4, 128), lambda i: (0, 0))],
        out_specs=pl.BlockSpec((4, be, 128), lambda i: (0, i, 0)),
        out_shape=_sds((4, ep, 128)))(rr, ss, mflat, cvec)


def _tc_update2(x, a0, a1, deg, wx, wa, wb, vmat, u2, ub2, br):
    n = x.shape[0]

    def body(x_ref, a0_ref, a1_ref, d_ref, wx_ref, wa_ref, wb_ref, vm_ref,
             u2_ref, ub2_ref, o_ref):
        xv = x_ref[...]
        wa_ = wa_ref[...]
        wb_ = wb_ref[...]
        t = (jnp.dot(xv, wx_ref[...], preferred_element_type=_F32)
             + jnp.dot(a0_ref[0], wa_[:64], preferred_element_type=_F32)
             + jnp.dot(a0_ref[1], wa_[64:], preferred_element_type=_F32)
             + jnp.dot(a1_ref[0], wb_[:64], preferred_element_type=_F32)
             + jnp.dot(a1_ref[1], wb_[64:], preferred_element_type=_F32)
             + jnp.dot(d_ref[...], vm_ref[...], preferred_element_type=_F32))
        o_ref[...] = xv + jnp.dot(_silu(t), u2_ref[...],
                                  preferred_element_type=_F32) + ub2_ref[...]

    return pl.pallas_call(
        body, grid=(n // br,),
        in_specs=[pl.BlockSpec((br, 128), lambda i: (i, 0)),
                  pl.BlockSpec((2, br, 64), lambda i: (0, i, 0)),
                  pl.BlockSpec((2, br, 64), lambda i: (0, i, 0)),
                  pl.BlockSpec((br, 128), lambda i: (i, 0)),
                  pl.BlockSpec((128, 128), lambda i: (0, 0)),
                  pl.BlockSpec((128, 128), lambda i: (0, 0)),
                  pl.BlockSpec((128, 128), lambda i: (0, 0)),
                  pl.BlockSpec((128, 128), lambda i: (0, 0)),
                  pl.BlockSpec((128, 128), lambda i: (0, 0)),
                  pl.BlockSpec((1, 128), lambda i: (0, 0))],
        out_specs=pl.BlockSpec((br, 128), lambda i: (i, 0)),
        out_shape=_sds((n, 128)))(x, a0, a1, deg, wx, wa, wb, vmat, u2,
                                  ub2.reshape(1, -1))


def _tc_update1(x, a0, deg, wx, wa, vmat, u2, ub2, br):
    n = x.shape[0]

    def body(x_ref, a0_ref, d_ref, wx_ref, wa_ref, vm_ref, u2_ref, ub2_ref,
             o_ref):
        xv = x_ref[...]
        wa_ = wa_ref[...]
        t = (jnp.dot(xv, wx_ref[...], preferred_element_type=_F32)
             + jnp.dot(a0_ref[0], wa_[:64], preferred_element_type=_F32)
             + jnp.dot(a0_ref[1], wa_[64:], preferred_element_type=_F32)
             + jnp.dot(d_ref[...], vm_ref[...], preferred_element_type=_F32))
        o_ref[...] = xv + jnp.dot(_silu(t), u2_ref[...],
                                  preferred_element_type=_F32) + ub2_ref[...]

    return pl.pallas_call(
        body, grid=(n // br,),
        in_specs=[pl.BlockSpec((br, 128), lambda i: (i, 0)),
                  pl.BlockSpec((2, br, 64), lambda i: (0, i, 0)),
                  pl.BlockSpec((br, 128), lambda i: (i, 0)),
                  pl.BlockSpec((128, 128), lambda i: (0, 0)),
                  pl.BlockSpec((128, 128), lambda i: (0, 0)),
                  pl.BlockSpec((128, 128), lambda i: (0, 0)),
                  pl.BlockSpec((128, 128), lambda i: (0, 0)),
                  pl.BlockSpec((1, 128), lambda i: (0, 0))],
        out_specs=pl.BlockSpec((br, 128), lambda i: (i, 0)),
        out_shape=_sds((n, 128)))(x, a0, deg, wx, wa, vmat, u2,
                                  ub2.reshape(1, -1))


def _tc_pool(y, ids3, br):
    n = y.shape[0]

    def body(y_ref, id_ref, o_ref):
        i = pl.program_id(0)

        @pl.when(i == 0)
        def _():
            o_ref[...] = jnp.zeros_like(o_ref)

        ids = id_ref[0]
        row = lax.broadcasted_iota(jnp.int32, (_B, br), 0)
        oh = (row == ids).astype(_F32)
        o_ref[...] += jnp.dot(oh, y_ref[...], preferred_element_type=_F32)

    return pl.pallas_call(
        body, grid=(n // br,),
        in_specs=[pl.BlockSpec((br, 128), lambda i: (i, 0)),
                  pl.BlockSpec((1, 1, br), lambda i: (i, 0, 0))],
        out_specs=pl.BlockSpec((_B, 128), lambda i: (0, 0)),
        out_shape=_sds((_B, 128)))(y, ids3)


def _tc_post(p0, p1, q1, qb1, q2, qb2):
    def body(p0_ref, p1_ref, q1_ref, qb1_ref, q2_ref, qb2_ref, o_ref):
        q1_ = q1_ref[...]
        t = (jnp.dot(p0_ref[...], q1_[:128], preferred_element_type=_F32)
             + jnp.dot(p1_ref[...], q1_[128:], preferred_element_type=_F32)
             + qb1_ref[...])
        o_ref[...] = (jnp.dot(_silu(t), q2_ref[...],
                              preferred_element_type=_F32) + qb2_ref[...])

    return pl.pallas_call(
        body,
        out_shape=_sds((_B, 128)))(p0, p1, q1, qb1.reshape(1, -1), q2,
                                   qb2.reshape(1, -1))


def _sc_gather16(table, idx, n_out):
    per = n_out // _NW
    nch = per // _CH
    mesh = plsc.VectorSubcoreMesh(core_axis_name="c", subcore_axis_name="s",
                                  num_cores=_NC, num_subcores=_NS)

    @functools.partial(
        pl.kernel,
        compiler_params=pltpu.CompilerParams(use_tc_tiling_on_sc=False),
        out_type=_sds((n_out, 16)), mesh=mesh,
        scratch_types=[pltpu.VMEM((_CH,), jnp.int32),
                       pltpu.VMEM((_CH, 16), _F32)])
    def k(t_hbm, i_hbm, o_hbm, idxv, rows):
        wid = lax.axis_index("s") * _NC + lax.axis_index("c")
        base = wid * per

        def chunk(j, _):
            off = base + j * _CH
            pltpu.sync_copy(i_hbm.at[pl.ds(off, _CH)], idxv)
            pltpu.sync_copy(t_hbm.at[idxv], rows)
            pltpu.sync_copy(rows, o_hbm.at[pl.ds(off, _CH)])
            return 0

        lax.fori_loop(0, nch, chunk, 0)

    return k(table, idx)


def _sc_prep(t0, t1, e00_0, e00_1, e01_0, e01_1, e11_0, e11_1,
             z16, ones2, n0, n1, etrues):
    ep00 = e00_0.shape[0]
    ep01 = e01_0.shape[0]
    ep11 = e11_0.shape[0]
    mesh = plsc.VectorSubcoreMesh(core_axis_name="c", subcore_axis_name="s",
                                  num_cores=_NC, num_subcores=_NS)
    nsl = 2

    out_type = [_sds((ep00, 16)), _sds((ep00, 16)),
                _sds((ep01, 16)), _sds((ep01, 16)),
                _sds((ep11, 16)), _sds((ep11, 16)),
                _sds((_NC, n0, 16)), _sds((_NC, n0, 16)),
                _sds((_NC, n1, 16))]

    scratch = ([pltpu.VMEM((_CH,), jnp.int32) for _ in range(nsl)]
               + [pltpu.VMEM((_CH, 16), _F32) for _ in range(2 * nsl)]
               + [pltpu.SemaphoreType.DMA for _ in range(4 * nsl)]
               + [pltpu.VMEM_SHARED((n0, 16), _F32),
                  pltpu.VMEM_SHARED((n0, 16), _F32),
                  pltpu.VMEM_SHARED((n1, 16), _F32)])

    @functools.partial(
        pl.kernel,
        compiler_params=pltpu.CompilerParams(use_tc_tiling_on_sc=False),
        out_type=out_type, mesh=mesh, scratch_types=scratch)
    def k(t0_hbm, t1_hbm, e000, e001, e010, e011, e110, e111,
          z16_hbm, ones2_hbm, r00, s00, r01, s01, r11, s11, d00o, d01o, d11o,
          *scr):
        idxv = scr[0:nsl]
        rows = scr[nsl:2 * nsl]
        dwin = scr[2 * nsl:3 * nsl]
        isem = scr[3 * nsl:4 * nsl]
        gsem = scr[4 * nsl:5 * nsl]
        wsem = scr[5 * nsl:6 * nsl]
        dsem = scr[6 * nsl:7 * nsl]
        d00 = scr[7 * nsl]
        d01 = scr[7 * nsl + 1]
        d11 = scr[7 * nsl + 2]
        c = lax.axis_index("c")
        s = lax.axis_index("s")
        wid = s * _NC + c

        for (tab, idx_hbm, out, ep) in ((t0_hbm, e000, r00, ep00),
                                        (t0_hbm, e001, s00, ep00),
                                        (t0_hbm, e010, r01, ep01),
                                        (t1_hbm, e011, s01, ep01),
                                        (t1_hbm, e110, r11, ep11),
                                        (t1_hbm, e111, s11, ep11)):
            per = ep // _NW
            base = wid * per
            nch = per // _CH

            def gwave(j0, nact):
                di = [pltpu.async_copy(idx_hbm.at[pl.ds(base + (j0 + sl) * _CH,
                                                        _CH)],
                                       idxv[sl], isem[sl])
                      for sl in range(nact)]
                dg = []
                for sl in range(nact):
                    di[sl].wait()
                    dg.append(pltpu.async_copy(tab.at[idxv[sl]], rows[sl],
                                               gsem[sl]))
                dw = []
                for sl in range(nact):
                    dg[sl].wait()
                    dw.append(pltpu.async_copy(
                        rows[sl],
                        out.at[pl.ds(base + (j0 + sl) * _CH, _CH)], wsem[sl]))
                for d in dw:
                    d.wait()

            def gbody(w, _):
                gwave(w * nsl, nsl)
                return 0
            lax.fori_loop(0, nch // nsl, gbody, 0)
            if nch % nsl:
                gwave((nch // nsl) * nsl, nch % nsl)

        for (acc, n) in ((d00, n0), (d01, n0), (d11, n1)):
            rows_per = n // _NS
            r0 = s * rows_per
            pltpu.sync_copy(z16_hbm.at[pl.ds(0, rows_per)],
                            acc.at[pl.ds(r0, rows_per)])
        plsc.subcore_barrier()

        for (idx_hbm, acc, ep, etrue) in ((e000, d00, ep00, etrues[0]),
                                          (e010, d01, ep01, etrues[1]),
                                          (e110, d11, ep11, etrues[2])):
            per = ep // _NW
            base = wid * per
            nch = per // _CH

            def dwave(j0, nact):
                di = [pltpu.async_copy(idx_hbm.at[pl.ds(base + (j0 + sl) * _CH,
                                                        _CH)],
                                       idxv[sl], isem[sl])
                      for sl in range(nact)]
                dg = []
                for sl in range(nact):
                    off = base + (j0 + sl) * _CH
                    nreal = jnp.clip(etrue - off, 0, _CH)
                    dg.append(pltpu.async_copy(
                        ones2_hbm.at[pl.ds(_CH - nreal, _CH)], dwin[sl],
                        gsem[sl]))
                dd = []
                for sl in range(nact):
                    di[sl].wait()
                    dg[sl].wait()
                    dd.append(pltpu.async_copy(dwin[sl], acc.at[idxv[sl]],
                                               dsem[sl], add=True))
                for d in dd:
                    d.wait()

            def dbody(w, _):
                dwave(w * nsl, nsl)
                return 0
            lax.fori_loop(0, nch // nsl, dbody, 0)
            if nch % nsl:
                dwave((nch // nsl) * nsl, nch % nsl)
        plsc.subcore_barrier()

        for (acc, out, n) in ((d00, d00o, n0), (d01, d01o, n0),
                              (d11, d11o, n1)):
            rows_per = n // _NS
            r0 = s * rows_per
            pltpu.sync_copy(acc.at[pl.ds(r0, rows_per)],
                            out.at[c, pl.ds(r0, rows_per)])

    return k(t0, t1, e00_0, e00_1, e01_0, e01_1, e11_0, e11_1, z16, ones2)


def _sc_edges(r00t, s00t, r01t, s01t, r11t, s11t,
              e00_0, e00_1, e01_0, e01_1, e11_0, e11_1,
              ip00, ip01, ip11, z64, layer_idx, n0, n1):
    ep00 = e00_0.shape[0]
    ep01 = e01_0.shape[0]
    ep11 = e11_0.shape[0]
    mesh = plsc.VectorSubcoreMesh(core_axis_name="c", subcore_axis_name="s",
                                  num_cores=_NC, num_subcores=_NS)
    nsl = 2

    out_type = [_sds((_NC, n0, 64)), _sds((_NC, n0, 64)),
                _sds((_NC, n1, 64))]

    scratch = ([pltpu.VMEM((_CH,), jnp.int32) for _ in range(3 * nsl)]
               + [pltpu.VMEM((_CH, 64), _F32) for _ in range(3 * nsl)]
               + [pltpu.SemaphoreType.DMA for _ in range(3 * nsl)]
               + [pltpu.VMEM_SHARED((n1, 64), _F32)])

    @functools.partial(
        pl.kernel,
        compiler_params=pltpu.CompilerParams(use_tc_tiling_on_sc=False),
        out_type=out_type, mesh=mesh, scratch_types=scratch)
    def k(tr00, ts00, tr01, ts01, tr11, ts11,
          e000, e001, e010, e011, e110, e111,
          i00, i01, i11, z64_hbm, a00o, a01o, a11o, *scr):
        idx0 = scr[0:nsl]
        idx0g = scr[nsl:2 * nsl]
        idx1g = scr[2 * nsl:3 * nsl]
        rbuf = scr[3 * nsl:4 * nsl]
        sbuf = scr[4 * nsl:5 * nsl]
        ibuf = scr[5 * nsl:6 * nsl]
        isem = scr[6 * nsl:7 * nsl]
        gsem = scr[7 * nsl:8 * nsl]
        ssem = scr[8 * nsl:9 * nsl]
        acc = scr[9 * nsl]
        c = lax.axis_index("c")
        s = lax.axis_index("s")

        phases = ((tr00, n0, ts00, n0, e000, e001, i00, a00o, n0, ep00),
                  (tr01, n0, ts01, n1, e010, e011, i01, a01o, n0, ep01),
                  (tr11, n1, ts11, n1, e110, e111, i11, a11o, n1, ep11))

        for (tr, nrt, ts, nst, e0h, e1h, iph, outh, nr, ep) in phases:
            rows_per = nr // _NS
            r0 = s * rows_per
            pltpu.sync_copy(z64_hbm.at[pl.ds(0, rows_per)],
                            acc.at[pl.ds(r0, rows_per)])
            plsc.subcore_barrier()

            per = ep // _NS
            base = s * per
            nch = per // _CH

            def issue_idx(j, sl):
                off = base + j * _CH
                d1 = pltpu.async_copy(e0h.at[pl.ds(off, _CH)], idx0[sl],
                                      isem[sl])
                d2 = pltpu.async_copy(e1h.at[pl.ds(off, _CH)], idx1g[sl],
                                      isem[sl])
                return d1, d2

            def issue_gathers(j, sl, idone):
                off = base + j * _CH
                for d in idone:
                    d.wait()

                for j4 in range(_CH // 16):
                    sl16 = pl.ds(j4 * 16, 16)
                    idx0g[sl][sl16] = idx0[sl][sl16] + c * nrt
                    idx1g[sl][sl16] = idx1g[sl][sl16] + c * nst
                d1 = pltpu.async_copy(tr.at[idx0g[sl]], rbuf[sl], gsem[sl])
                d2 = pltpu.async_copy(ts.at[idx1g[sl]], sbuf[sl], gsem[sl])
                d3 = pltpu.async_copy(
                    iph.at[layer_idx, pl.ds(off, _CH), pl.ds(c * 64, 64)],
                    ibuf[sl], gsem[sl])
                return d1, d2, d3

            def compute_scatter(sl, gdone):
                for d in gdone:
                    d.wait()

                @plsc.parallel_loop(0, _CH, unroll=8)
                def _(i):
                    for j4 in range(4):
                        sl16 = pl.ds(j4 * 16, 16)
                        v = (rbuf[sl][i, sl16] + sbuf[sl][i, sl16]
                             + ibuf[sl][i, sl16])
                        rbuf[sl][i, sl16] = v / (1.0 + jnp.exp(-v))
                return pltpu.async_copy(rbuf[sl], acc.at[idx0[sl]],
                                        ssem[sl], add=True)

            def wave(j0, nact):
                idone = [issue_idx(j0 + sl, sl) for sl in range(nact)]
                gdone = [issue_gathers(j0 + sl, sl, idone[sl])
                         for sl in range(nact)]
                sdone = [compute_scatter(sl, gdone[sl])
                         for sl in range(nact)]
                for d in sdone:
                    d.wait()

            nwave = nch // nsl

            def wbody(w, _):
                wave(w * nsl, nsl)
                return 0
            lax.fori_loop(0, nwave, wbody, 0)
            if nch % nsl:
                wave(nwave * nsl, nch % nsl)
            plsc.subcore_barrier()

            pltpu.sync_copy(acc.at[pl.ds(r0, rows_per)],
                            outh.at[c, pl.ds(r0, rows_per)])
            plsc.subcore_barrier()

    return k(r00t, s00t, r01t, s01t, r11t, s11t,
             e00_0, e00_1, e01_0, e01_1, e11_0, e11_1,
             ip00, ip01, ip11, z64)


def kernel(pos, x_0, x_1, cell_1, adj_0_0, adj_0_1, adj_1_1,
           cell_0_batch, cell_1_batch, batch_size, params):
    n0 = pos.shape[0]
    n1 = x_1.shape[0]
    hid = x_0.shape[1]
    e00 = adj_0_0.shape[1]
    e01 = adj_0_1.shape[1]
    e11 = adj_1_1.shape[1]
    ep00 = _rup(e00, 4096)
    ep01 = _rup(e01, 4096)
    ep11 = _rup(e11, 4096)
    n1p = _rup(n1, 4096)

    def padi(v, n):
        return jnp.pad(v.astype(jnp.int32), (0, n - v.shape[0]))

    t0 = jnp.concatenate([pos, jnp.zeros((n0, 13), _F32)], axis=1)

    i0 = padi(cell_1[:, 0], n1p)
    i1 = padi(cell_1[:, 1], n1p)
    ra = _sc_gather16(t0, i0, n1p)
    rb = _sc_gather16(t0, i1, n1p)
    t1 = _tc_t1(ra, rb, 512)

    e00_0 = padi(adj_0_0[0], ep00)
    e00_1 = padi(adj_0_0[1], ep00)
    e01_0 = padi(adj_0_1[0], ep01)
    e01_1 = padi(adj_0_1[1], ep01)
    e11_0 = padi(adj_1_1[0], ep11)
    e11_1 = padi(adj_1_1[1], ep11)

    z16 = jnp.zeros((n1 // _NS, 16), _F32)
    z64 = jnp.zeros((n1 // _NS, 64), _F32)
    ones2 = jnp.concatenate([jnp.ones((_CH, 16), _F32),
                             jnp.zeros((_CH, 16), _F32)], axis=0)

    (r00, s00, r01, s01, r11, s11, d00p, d01p, d11p) = _sc_prep(
        t0, t1, e00_0, e00_1, e01_0, e01_1, e11_0, e11_1,
        z16, ones2, n0, n1, (e00, e01, e11))

    stats = {"0_0": _tc_stats(r00, s00, e00, 2048),
             "0_1": _tc_stats(r01, s01, e01, 2048),
             "1_1": _tc_stats(r11, s11, e11, 2048)}
    scales, shifts = {}, {}
    for a, ecnt in (("0_0", e00), ("0_1", e01), ("1_1", e11)):
        sm = stats[a][0, :3] / ecnt
        sq = stats[a][0, 3:6] / ecnt
        var = sq - sm * sm
        g, bt = params["bn"][a]
        sc = g / jnp.sqrt(var + 1e-5)
        scales[a] = sc
        shifts[a] = bt - sm * sc

    layers = params["layers"]
    minv, cinv = {}, {}
    for a in ("0_0", "0_1", "1_1"):
        ms, cs = [], []
        for layer in layers:
            w1, b1 = layer["msg_" + a][0]
            w1i = w1[2 * hid:]
            ms.append(scales[a][:, None] * w1i)
            cs.append(shifts[a] @ w1i + b1)
        minv[a] = jnp.concatenate(ms, axis=0)
        cinv[a] = jnp.stack(cs, axis=0)

    ip00 = _tc_invp(r00, s00, minv["0_0"], cinv["0_0"], e00, 2048)
    ip01 = _tc_invp(r01, s01, minv["0_1"], cinv["0_1"], e01, 2048)
    ip11 = _tc_invp(r11, s11, minv["1_1"], cinv["1_1"], e11, 2048)

    d00 = d00p[0, :, 0] + d00p[1, :, 0]
    d01 = d01p[0, :, 0] + d01p[1, :, 0]
    d11 = d11p[0, :, 0] + d11p[1, :, 0]
    deg0 = jnp.concatenate([d00[:, None], d01[:, None],
                            jnp.ones((n0, 1), _F32),
                            jnp.zeros((n0, 125), _F32)], axis=1)
    deg1 = jnp.concatenate([d11[:, None], jnp.ones((n1, 1), _F32),
                            jnp.zeros((n1, 126), _F32)], axis=1)

    x0 = _tc_dense(x_0, params["emb_0"][0], params["emb_0"][1], 400)
    x1 = _tc_dense(x_1, params["emb_1"][0], params["emb_1"][1], 800)

    for li, layer in enumerate(layers):
        w00 = layer["msg_0_0"][0][0]
        w01 = layer["msg_0_1"][0][0]
        w11 = layer["msg_1_1"][0][0]
        wcat0 = jnp.concatenate([w00[:hid], w00[hid:2 * hid], w01[:hid]], axis=1)
        wcat1 = jnp.concatenate([w01[hid:2 * hid], w11[:hid], w11[hid:2 * hid]],
                                axis=1)
        p0a, p0b, p0c = _tc_proj(x0, wcat0, 400)
        p1a, p1b, p1c = _tc_proj(x1, wcat1, 800)
        r00t, s00t, r01t = (p.reshape(2 * n0, 64) for p in (p0a, p0b, p0c))
        s01t, r11t, s11t = (p.reshape(2 * n1, 64) for p in (p1a, p1b, p1c))

        a00, a01, a11 = _sc_edges(r00t, s00t, r01t, s01t, r11t, s11t,
                                  e00_0, e00_1, e01_0, e01_1, e11_0, e11_1,
                                  ip00, ip01, ip11, z64, li, n0, n1)

        (u1, ub1), (u2, ub2) = layer["upd_0"]
        w2_00, b2_00 = layer["msg_0_0"][1]
        w2_01, b2_01 = layer["msg_0_1"][1]
        u1x, u1a, u1b = u1[:hid], u1[hid:2 * hid], u1[2 * hid:]
        a00m = w2_00 @ u1a
        a01m = w2_01 @ u1b
        vmat0 = jnp.concatenate([(b2_00 @ u1a)[None], (b2_01 @ u1b)[None],
                                 ub1[None], jnp.zeros((125, 128), _F32)],
                                axis=0)
        x0 = _tc_update2(x0, a00, a01, deg0, u1x, a00m, a01m, vmat0, u2, ub2,
                         400)

        (v1, vb1), (v2, vb2) = layer["upd_1"]
        w2_11, b2_11 = layer["msg_1_1"][1]
        v1x, v1c = v1[:hid], v1[hid:]
        a11m = w2_11 @ v1c
        vmat1 = jnp.concatenate([(b2_11 @ v1c)[None], vb1[None],
                                 jnp.zeros((126, 128), _F32)], axis=0)
        x1 = _tc_update1(x1, a11, deg1, v1x, a11m, vmat1, v2, vb2, 800)

    y0 = _tc_mlp2(x0, params["pre_0"][0][0], params["pre_0"][0][1],
                  params["pre_0"][1][0], params["pre_0"][1][1], 400)
    y1 = _tc_mlp2(x1, params["pre_1"][0][0], params["pre_1"][0][1],
                  params["pre_1"][1][0], params["pre_1"][1][1], 800)

    ids0 = cell_0_batch.astype(jnp.int32).reshape(n0 // 400, 1, 400)
    ids1 = cell_1_batch.astype(jnp.int32).reshape(n1 // 800, 1, 800)
    p0 = _tc_pool(y0, ids0, 400)
    p1 = _tc_pool(y1, ids1, 800)

    out = _tc_post(p0, p1, params["post"][0][0], params["post"][0][1],
                   params["post"][1][0], params["post"][1][1])
    return jnp.squeeze(out)


# speedup vs baseline: 1.0545x; 1.0545x over previous
import functools

import jax
import jax.numpy as jnp
from jax import lax
from jax.experimental import pallas as pl
from jax.experimental.pallas import tpu as pltpu
from jax.experimental.pallas import tpu_sc as plsc

_B = 100
_NC, _NS = 2, 16
_NW = _NC * _NS
_CH = 128
_F32 = jnp.float32


def _sds(shape, dtype=_F32):
    return jax.ShapeDtypeStruct(shape, dtype)


def _rup(n, m):
    return ((n + m - 1) // m) * m


def _silu(x):
    return x / (1.0 + jnp.exp(-x))


def _tc_dense(x, w, b, br):
    n, k = x.shape
    ko = w.shape[1]

    def body(x_ref, w_ref, b_ref, o_ref):
        o_ref[...] = (jnp.dot(x_ref[...], w_ref[...],
                              preferred_element_type=_F32) + b_ref[...])

    return pl.pallas_call(
        body, grid=(n // br,),
        in_specs=[pl.BlockSpec((br, k), lambda i: (i, 0)),
                  pl.BlockSpec((k, ko), lambda i: (0, 0)),
                  pl.BlockSpec((1, ko), lambda i: (0, 0))],
        out_specs=pl.BlockSpec((br, ko), lambda i: (i, 0)),
        out_shape=_sds((n, ko)))(x, w, b.reshape(1, -1))


def _tc_mlp2(x, w1, b1, w2, b2, br):
    n, k = x.shape
    km = w1.shape[1]
    ko = w2.shape[1]

    def body(x_ref, w1_ref, b1_ref, w2_ref, b2_ref, o_ref):
        t = jnp.dot(x_ref[...], w1_ref[...], preferred_element_type=_F32) + b1_ref[...]
        o_ref[...] = (jnp.dot(_silu(t), w2_ref[...],
                              preferred_element_type=_F32) + b2_ref[...])

    return pl.pallas_call(
        body, grid=(n // br,),
        in_specs=[pl.BlockSpec((br, k), lambda i: (i, 0)),
                  pl.BlockSpec((k, km), lambda i: (0, 0)),
                  pl.BlockSpec((1, km), lambda i: (0, 0)),
                  pl.BlockSpec((km, ko), lambda i: (0, 0)),
                  pl.BlockSpec((1, ko), lambda i: (0, 0))],
        out_specs=pl.BlockSpec((br, ko), lambda i: (i, 0)),
        out_shape=_sds((n, ko)))(x, w1, b1.reshape(1, -1), w2, b2.reshape(1, -1))


def _tc_proj(x, wcat, br):
    n = x.shape[0]

    def body(x_ref, w_ref, o0_ref, o1_ref, o2_ref):
        y = jnp.dot(x_ref[...], w_ref[...], preferred_element_type=_F32)
        for t, o_ref in enumerate((o0_ref, o1_ref, o2_ref)):
            for c in range(2):
                o_ref[c] = y[:, t * 128 + c * 64: t * 128 + (c + 1) * 64]

    return pl.pallas_call(
        body, grid=(n // br,),
        in_specs=[pl.BlockSpec((br, 128), lambda i: (i, 0)),
                  pl.BlockSpec((128, 384), lambda i: (0, 0))],
        out_specs=[pl.BlockSpec((2, br, 64), lambda i: (0, i, 0))] * 3,
        out_shape=[_sds((2, n, 64))] * 3)(x, wcat)


def _tc_t1(a, b, br):
    n = a.shape[0]

    def body(a_ref, b_ref, o_ref):
        av = a_ref[...]
        bv = b_ref[...]
        c = 0.5 * (av + bv)
        d2 = jnp.sum((av - bv) ** 2, axis=1, keepdims=True)
        d = jnp.sqrt(d2 + 1e-12)
        lane = lax.broadcasted_iota(jnp.int32, (br, 16), 1)
        o_ref[...] = jnp.where(lane < 3, c, 0.0) + jnp.where(lane == 3, d, 0.0)

    return pl.pallas_call(
        body, grid=(n // br,),
        in_specs=[pl.BlockSpec((br, 16), lambda i: (i, 0)),
                  pl.BlockSpec((br, 16), lambda i: (i, 0))],
        out_specs=pl.BlockSpec((br, 16), lambda i: (i, 0)),
        out_shape=_sds((n, 16)))(a, b)


def _tc_stats(rr, ss, etrue, be):
    ep = rr.shape[0]

    def body(r_ref, s_ref, o_ref):
        i = pl.program_id(0)

        @pl.when(i == 0)
        def _():
            o_ref[...] = jnp.zeros_like(o_ref)

        a = r_ref[...]
        b = s_ref[...]
        lane = lax.broadcasted_iota(jnp.int32, (be, 16), 1)
        sub = lax.broadcasted_iota(jnp.int32, (be, 16), 0)
        valid = (sub + i * be) < etrue
        d = jnp.where((lane < 3) & valid, a - b, 0.0)
        dd = jnp.sum(d * d, axis=1, keepdims=True)
        dr = jnp.sum(jnp.where((lane == 3) & valid, a, 0.0), axis=1, keepdims=True)
        ds = jnp.sum(jnp.where((lane == 3) & valid, b, 0.0), axis=1, keepdims=True)
        vals = [jnp.sum(dd), jnp.sum(dr), jnp.sum(ds),
                jnp.sum(dd * dd), jnp.sum(dr * dr), jnp.sum(ds * ds)]
        l2 = lax.broadcasted_iota(jnp.int32, (8, 128), 1)
        s2 = lax.broadcasted_iota(jnp.int32, (8, 128), 0)
        p = jnp.zeros((8, 128), _F32)
        for k, v in enumerate(vals):
            p = p + jnp.where((s2 == 0) & (l2 == k), v, 0.0)
        o_ref[...] += p

    return pl.pallas_call(
        body, grid=(ep // be,),
        in_specs=[pl.BlockSpec((be, 16), lambda i: (i, 0)),
                  pl.BlockSpec((be, 16), lambda i: (i, 0))],
        out_specs=pl.BlockSpec((8, 128), lambda i: (0, 0)),
        out_shape=_sds((8, 128)))(rr, ss)


def _tc_invp(rr, ss, mflat, cvec, etrue, be):
    ep = rr.shape[0]

    def body(r_ref, s_ref, m_ref, c_ref, o_ref):
        i = pl.program_id(0)
        a = r_ref[...]
        b = s_ref[...]
        lane = lax.broadcasted_iota(jnp.int32, (be, 16), 1)
        d = jnp.where(lane < 3, a - b, 0.0)
        dd = jnp.sum(d * d, axis=1, keepdims=True)
        dr = a[:, 3:4]
        dsv = b[:, 3:4]
        sub = lax.broadcasted_iota(jnp.int32, (be, 1), 0)
        pad = (sub + i * be) >= etrue
        m = m_ref[...]
        cv = c_ref[...]
        for l in range(4):
            y = (dd * m[3 * l + 0:3 * l + 1, :] + dr * m[3 * l + 1:3 * l + 2, :]
                 + dsv * m[3 * l + 2:3 * l + 3, :] + cv[l:l + 1, :])
            o_ref[l] = jnp.where(pad, -1e4, y)

    return pl.pallas_call(
        body, grid=(ep // be,),
        in_specs=[pl.BlockSpec((be, 16), lambda i: (i, 0)),
                  pl.BlockSpec((be, 16), lambda i: (i, 0)),
                  pl.BlockSpec((12, 128), lambda i: (0, 0)),
                  pl.BlockSpec((4, 128), lambda i: (0, 0))],
        out_specs=pl.BlockSpec((4, be, 128), lambda i: (0, i, 0)),
        out_shape=_sds((4, ep, 128)))(rr, ss, mflat, cvec)


def _tc_update2(x, a0, a1, deg, wx, wa, wb, vmat, u2, ub2, br):
    n = x.shape[0]

    def body(x_ref, a0_ref, a1_ref, d_ref, wx_ref, wa_ref, wb_ref, vm_ref,
             u2_ref, ub2_ref, o_ref):
        xv = x_ref[...]
        wa_ = wa_ref[...]
        wb_ = wb_ref[...]
        t = (jnp.dot(xv, wx_ref[...], preferred_element_type=_F32)
             + jnp.dot(a0_ref[0], wa_[:64], preferred_element_type=_F32)
             + jnp.dot(a0_ref[1], wa_[64:], preferred_element_type=_F32)
             + jnp.dot(a1_ref[0], wb_[:64], preferred_element_type=_F32)
             + jnp.dot(a1_ref[1], wb_[64:], preferred_element_type=_F32)
             + jnp.dot(d_ref[...], vm_ref[...], preferred_element_type=_F32))
        o_ref[...] = xv + jnp.dot(_silu(t), u2_ref[...],
                                  preferred_element_type=_F32) + ub2_ref[...]

    return pl.pallas_call(
        body, grid=(n // br,),
        in_specs=[pl.BlockSpec((br, 128), lambda i: (i, 0)),
                  pl.BlockSpec((2, br, 64), lambda i: (0, i, 0)),
                  pl.BlockSpec((2, br, 64), lambda i: (0, i, 0)),
                  pl.BlockSpec((br, 128), lambda i: (i, 0)),
                  pl.BlockSpec((128, 128), lambda i: (0, 0)),
                  pl.BlockSpec((128, 128), lambda i: (0, 0)),
                  pl.BlockSpec((128, 128), lambda i: (0, 0)),
                  pl.BlockSpec((128, 128), lambda i: (0, 0)),
                  pl.BlockSpec((128, 128), lambda i: (0, 0)),
                  pl.BlockSpec((1, 128), lambda i: (0, 0))],
        out_specs=pl.BlockSpec((br, 128), lambda i: (i, 0)),
        out_shape=_sds((n, 128)))(x, a0, a1, deg, wx, wa, wb, vmat, u2,
                                  ub2.reshape(1, -1))


def _tc_update1(x, a0, deg, wx, wa, vmat, u2, ub2, br):
    n = x.shape[0]

    def body(x_ref, a0_ref, d_ref, wx_ref, wa_ref, vm_ref, u2_ref, ub2_ref,
             o_ref):
        xv = x_ref[...]
        wa_ = wa_ref[...]
        t = (jnp.dot(xv, wx_ref[...], preferred_element_type=_F32)
             + jnp.dot(a0_ref[0], wa_[:64], preferred_element_type=_F32)
             + jnp.dot(a0_ref[1], wa_[64:], preferred_element_type=_F32)
             + jnp.dot(d_ref[...], vm_ref[...], preferred_element_type=_F32))
        o_ref[...] = xv + jnp.dot(_silu(t), u2_ref[...],
                                  preferred_element_type=_F32) + ub2_ref[...]

    return pl.pallas_call(
        body, grid=(n // br,),
        in_specs=[pl.BlockSpec((br, 128), lambda i: (i, 0)),
                  pl.BlockSpec((2, br, 64), lambda i: (0, i, 0)),
                  pl.BlockSpec((br, 128), lambda i: (i, 0)),
                  pl.BlockSpec((128, 128), lambda i: (0, 0)),
                  pl.BlockSpec((128, 128), lambda i: (0, 0)),
                  pl.BlockSpec((128, 128), lambda i: (0, 0)),
                  pl.BlockSpec((128, 128), lambda i: (0, 0)),
                  pl.BlockSpec((1, 128), lambda i: (0, 0))],
        out_specs=pl.BlockSpec((br, 128), lambda i: (i, 0)),
        out_shape=_sds((n, 128)))(x, a0, deg, wx, wa, vmat, u2,
                                  ub2.reshape(1, -1))


def _tc_pool(y, ids3, br):
    n = y.shape[0]

    def body(y_ref, id_ref, o_ref):
        i = pl.program_id(0)

        @pl.when(i == 0)
        def _():
            o_ref[...] = jnp.zeros_like(o_ref)

        ids = id_ref[0]
        row = lax.broadcasted_iota(jnp.int32, (_B, br), 0)
        oh = (row == ids).astype(_F32)
        o_ref[...] += jnp.dot(oh, y_ref[...], preferred_element_type=_F32)

    return pl.pallas_call(
        body, grid=(n // br,),
        in_specs=[pl.BlockSpec((br, 128), lambda i: (i, 0)),
                  pl.BlockSpec((1, 1, br), lambda i: (i, 0, 0))],
        out_specs=pl.BlockSpec((_B, 128), lambda i: (0, 0)),
        out_shape=_sds((_B, 128)))(y, ids3)


def _tc_post(p0, p1, q1, qb1, q2, qb2):
    def body(p0_ref, p1_ref, q1_ref, qb1_ref, q2_ref, qb2_ref, o_ref):
        q1_ = q1_ref[...]
        t = (jnp.dot(p0_ref[...], q1_[:128], preferred_element_type=_F32)
             + jnp.dot(p1_ref[...], q1_[128:], preferred_element_type=_F32)
             + qb1_ref[...])
        o_ref[...] = (jnp.dot(_silu(t), q2_ref[...],
                              preferred_element_type=_F32) + qb2_ref[...])

    return pl.pallas_call(
        body,
        out_shape=_sds((_B, 128)))(p0, p1, q1, qb1.reshape(1, -1), q2,
                                   qb2.reshape(1, -1))


def _sc_gather16(table, idx, n_out):
    per = n_out // _NW
    nch = per // _CH
    mesh = plsc.VectorSubcoreMesh(core_axis_name="c", subcore_axis_name="s",
                                  num_cores=_NC, num_subcores=_NS)

    @functools.partial(
        pl.kernel,
        compiler_params=pltpu.CompilerParams(use_tc_tiling_on_sc=False),
        out_type=_sds((n_out, 16)), mesh=mesh,
        scratch_types=[pltpu.VMEM((_CH,), jnp.int32),
                       pltpu.VMEM((_CH, 16), _F32)])
    def k(t_hbm, i_hbm, o_hbm, idxv, rows):
        wid = lax.axis_index("s") * _NC + lax.axis_index("c")
        base = wid * per

        def chunk(j, _):
            off = base + j * _CH
            pltpu.sync_copy(i_hbm.at[pl.ds(off, _CH)], idxv)
            pltpu.sync_copy(t_hbm.at[idxv], rows)
            pltpu.sync_copy(rows, o_hbm.at[pl.ds(off, _CH)])
            return 0

        lax.fori_loop(0, nch, chunk, 0)

    return k(table, idx)


def _sc_prep(t0, t1, e00_0, e00_1, e01_0, e01_1, e11_0, e11_1,
             z16, ones2, n0, n1, etrues):
    ep00 = e00_0.shape[0]
    ep01 = e01_0.shape[0]
    ep11 = e11_0.shape[0]
    mesh = plsc.VectorSubcoreMesh(core_axis_name="c", subcore_axis_name="s",
                                  num_cores=_NC, num_subcores=_NS)
    nsl = 2

    out_type = [_sds((ep00, 16)), _sds((ep00, 16)),
                _sds((ep01, 16)), _sds((ep01, 16)),
                _sds((ep11, 16)), _sds((ep11, 16)),
                _sds((_NC, n0, 16)), _sds((_NC, n0, 16)),
                _sds((_NC, n1, 16))]

    scratch = ([pltpu.VMEM((_CH,), jnp.int32) for _ in range(nsl)]
               + [pltpu.VMEM((_CH, 16), _F32) for _ in range(2 * nsl)]
               + [pltpu.SemaphoreType.DMA for _ in range(4 * nsl)]
               + [pltpu.VMEM_SHARED((n0, 16), _F32),
                  pltpu.VMEM_SHARED((n0, 16), _F32),
                  pltpu.VMEM_SHARED((n1, 16), _F32)])

    @functools.partial(
        pl.kernel,
        compiler_params=pltpu.CompilerParams(use_tc_tiling_on_sc=False),
        out_type=out_type, mesh=mesh, scratch_types=scratch)
    def k(t0_hbm, t1_hbm, e000, e001, e010, e011, e110, e111,
          z16_hbm, ones2_hbm, r00, s00, r01, s01, r11, s11, d00o, d01o, d11o,
          *scr):
        idxv = scr[0:nsl]
        rows = scr[nsl:2 * nsl]
        dwin = scr[2 * nsl:3 * nsl]
        isem = scr[3 * nsl:4 * nsl]
        gsem = scr[4 * nsl:5 * nsl]
        wsem = scr[5 * nsl:6 * nsl]
        dsem = scr[6 * nsl:7 * nsl]
        d00 = scr[7 * nsl]
        d01 = scr[7 * nsl + 1]
        d11 = scr[7 * nsl + 2]
        c = lax.axis_index("c")
        s = lax.axis_index("s")
        wid = s * _NC + c

        for (tab, idx_hbm, out, ep) in ((t0_hbm, e000, r00, ep00),
                                        (t0_hbm, e001, s00, ep00),
                                        (t0_hbm, e010, r01, ep01),
                                        (t1_hbm, e011, s01, ep01),
                                        (t1_hbm, e110, r11, ep11),
                                        (t1_hbm, e111, s11, ep11)):
            per = ep // _NW
            base = wid * per
            nch = per // _CH

            def gwave(j0, nact):
                di = [pltpu.async_copy(idx_hbm.at[pl.ds(base + (j0 + sl) * _CH,
                                                        _CH)],
                                       idxv[sl], isem[sl])
                      for sl in range(nact)]
                dg = []
                for sl in range(nact):
                    di[sl].wait()
                    dg.append(pltpu.async_copy(tab.at[idxv[sl]], rows[sl],
                                               gsem[sl]))
                dw = []
                for sl in range(nact):
                    dg[sl].wait()
                    dw.append(pltpu.async_copy(
                        rows[sl],
                        out.at[pl.ds(base + (j0 + sl) * _CH, _CH)], wsem[sl]))
                for d in dw:
                    d.wait()

            def gbody(w, _):
                gwave(w * nsl, nsl)
                return 0
            lax.fori_loop(0, nch // nsl, gbody, 0)
            if nch % nsl:
                gwave((nch // nsl) * nsl, nch % nsl)

        for (acc, n) in ((d00, n0), (d01, n0), (d11, n1)):
            rows_per = n // _NS
            r0 = s * rows_per
            pltpu.sync_copy(z16_hbm.at[pl.ds(0, rows_per)],
                            acc.at[pl.ds(r0, rows_per)])
        plsc.subcore_barrier()

        for (idx_hbm, acc, ep, etrue) in ((e000, d00, ep00, etrues[0]),
                                          (e010, d01, ep01, etrues[1]),
                                          (e110, d11, ep11, etrues[2])):
            per = ep // _NW
            base = wid * per
            nch = per // _CH

            def dwave(j0, nact):
                di = [pltpu.async_copy(idx_hbm.at[pl.ds(base + (j0 + sl) * _CH,
                                                        _CH)],
                                       idxv[sl], isem[sl])
                      for sl in range(nact)]
                dg = []
                for sl in range(nact):
                    off = base + (j0 + sl) * _CH
                    nreal = jnp.clip(etrue - off, 0, _CH)
                    dg.append(pltpu.async_copy(
                        ones2_hbm.at[pl.ds(_CH - nreal, _CH)], dwin[sl],
                        gsem[sl]))
                dd = []
                for sl in range(nact):
                    di[sl].wait()
                    dg[sl].wait()
                    dd.append(pltpu.async_copy(dwin[sl], acc.at[idxv[sl]],
                                               dsem[sl], add=True))
                for d in dd:
                    d.wait()

            def dbody(w, _):
                dwave(w * nsl, nsl)
                return 0
            lax.fori_loop(0, nch // nsl, dbody, 0)
            if nch % nsl:
                dwave((nch // nsl) * nsl, nch % nsl)
        plsc.subcore_barrier()

        for (acc, out, n) in ((d00, d00o, n0), (d01, d01o, n0),
                              (d11, d11o, n1)):
            rows_per = n // _NS
            r0 = s * rows_per
            pltpu.sync_copy(acc.at[pl.ds(r0, rows_per)],
                            out.at[c, pl.ds(r0, rows_per)])

    return k(t0, t1, e00_0, e00_1, e01_0, e01_1, e11_0, e11_1, z16, ones2)


def _sc_edges(r00t, s00t, r01t, s01t, r11t, s11t,
              e00_0, e00_1, e01_0, e01_1, e11_0, e11_1,
              ip00, ip01, ip11, z64, layer_idx, n0, n1):
    ep00 = e00_0.shape[0]
    ep01 = e01_0.shape[0]
    ep11 = e11_0.shape[0]
    mesh = plsc.VectorSubcoreMesh(core_axis_name="c", subcore_axis_name="s",
                                  num_cores=_NC, num_subcores=_NS)
    nsl = 2

    out_type = [_sds((_NC, n0, 64)), _sds((_NC, n0, 64)),
                _sds((_NC, n1, 64))]

    scratch = ([pltpu.VMEM((_CH,), jnp.int32) for _ in range(3 * nsl)]
               + [pltpu.VMEM((_CH, 64), _F32) for _ in range(3 * nsl)]
               + [pltpu.SemaphoreType.DMA for _ in range(3 * nsl)]
               + [pltpu.VMEM_SHARED((n1, 64), _F32)])

    @functools.partial(
        pl.kernel,
        compiler_params=pltpu.CompilerParams(use_tc_tiling_on_sc=False),
        out_type=out_type, mesh=mesh, scratch_types=scratch)
    def k(tr00, ts00, tr01, ts01, tr11, ts11,
          e000, e001, e010, e011, e110, e111,
          i00, i01, i11, z64_hbm, a00o, a01o, a11o, *scr):
        idx0 = scr[0:nsl]
        idx0g = scr[nsl:2 * nsl]
        idx1g = scr[2 * nsl:3 * nsl]
        rbuf = scr[3 * nsl:4 * nsl]
        sbuf = scr[4 * nsl:5 * nsl]
        ibuf = scr[5 * nsl:6 * nsl]
        isem = scr[6 * nsl:7 * nsl]
        gsem = scr[7 * nsl:8 * nsl]
        ssem = scr[8 * nsl:9 * nsl]
        acc = scr[9 * nsl]
        c = lax.axis_index("c")
        s = lax.axis_index("s")

        phases = ((tr00, n0, ts00, n0, e000, e001, i00, a00o, n0, ep00),
                  (tr01, n0, ts01, n1, e010, e011, i01, a01o, n0, ep01),
                  (tr11, n1, ts11, n1, e110, e111, i11, a11o, n1, ep11))

        for (tr, nrt, ts, nst, e0h, e1h, iph, outh, nr, ep) in phases:
            rows_per = nr // _NS
            r0 = s * rows_per
            pltpu.sync_copy(z64_hbm.at[pl.ds(0, rows_per)],
                            acc.at[pl.ds(r0, rows_per)])
            plsc.subcore_barrier()

            per = ep // _NS
            base = s * per
            nch = per // _CH

            def issue_idx(j, sl):
                off = base + j * _CH
                d1 = pltpu.async_copy(e0h.at[pl.ds(off, _CH)], idx0[sl],
                                      isem[sl])
                d2 = pltpu.async_copy(e1h.at[pl.ds(off, _CH)], idx1g[sl],
                                      isem[sl])
                d3 = pltpu.async_copy(
                    iph.at[layer_idx, pl.ds(off, _CH), pl.ds(c * 64, 64)],
                    ibuf[sl], gsem[sl])
                return d1, d2, d3

            def issue_gathers(j, sl, idone):
                d3 = idone[2]
                for d in idone[:2]:
                    d.wait()

                for j4 in range(_CH // 16):
                    sl16 = pl.ds(j4 * 16, 16)
                    idx0g[sl][sl16] = idx0[sl][sl16] + c * nrt
                    idx1g[sl][sl16] = idx1g[sl][sl16] + c * nst
                d1 = pltpu.async_copy(tr.at[idx0g[sl]], rbuf[sl], gsem[sl])
                d2 = pltpu.async_copy(ts.at[idx1g[sl]], sbuf[sl], gsem[sl])
                return d1, d2, d3

            def compute_scatter(sl, gdone):
                for d in gdone:
                    d.wait()

                def row(i8, _):
                    for r8 in range(8):
                        i = i8 * 8 + r8
                        for j4 in range(4):
                            sl16 = pl.ds(j4 * 16, 16)
                            v = (rbuf[sl][i, sl16] + sbuf[sl][i, sl16]
                                 + ibuf[sl][i, sl16])
                            rbuf[sl][i, sl16] = v / (1.0 + jnp.exp(-v))
                    return 0
                lax.fori_loop(0, _CH // 8, row, 0)
                return pltpu.async_copy(rbuf[sl], acc.at[idx0[sl]],
                                        ssem[sl], add=True)

            def wave(j0, nact):
                idone = [issue_idx(j0 + sl, sl) for sl in range(nact)]
                gdone = [issue_gathers(j0 + sl, sl, idone[sl])
                         for sl in range(nact)]
                sdone = [compute_scatter(sl, gdone[sl])
                         for sl in range(nact)]
                for d in sdone:
                    d.wait()

            nwave = nch // nsl

            def wbody(w, _):
                wave(w * nsl, nsl)
                return 0
            lax.fori_loop(0, nwave, wbody, 0)
            if nch % nsl:
                wave(nwave * nsl, nch % nsl)
            plsc.subcore_barrier()

            pltpu.sync_copy(acc.at[pl.ds(r0, rows_per)],
                            outh.at[c, pl.ds(r0, rows_per)])
            plsc.subcore_barrier()

    return k(r00t, s00t, r01t, s01t, r11t, s11t,
             e00_0, e00_1, e01_0, e01_1, e11_0, e11_1,
             ip00, ip01, ip11, z64)


def kernel(pos, x_0, x_1, cell_1, adj_0_0, adj_0_1, adj_1_1,
           cell_0_batch, cell_1_batch, batch_size, params):
    n0 = pos.shape[0]
    n1 = x_1.shape[0]
    hid = x_0.shape[1]
    e00 = adj_0_0.shape[1]
    e01 = adj_0_1.shape[1]
    e11 = adj_1_1.shape[1]
    ep00 = _rup(e00, 4096)
    ep01 = _rup(e01, 4096)
    ep11 = _rup(e11, 4096)
    n1p = _rup(n1, 4096)

    def padi(v, n):
        return jnp.pad(v.astype(jnp.int32), (0, n - v.shape[0]))

    t0 = jnp.concatenate([pos, jnp.zeros((n0, 13), _F32)], axis=1)

    i0 = padi(cell_1[:, 0], n1p)
    i1 = padi(cell_1[:, 1], n1p)
    ra = _sc_gather16(t0, i0, n1p)
    rb = _sc_gather16(t0, i1, n1p)
    t1 = _tc_t1(ra, rb, 512)

    e00_0 = padi(adj_0_0[0], ep00)
    e00_1 = padi(adj_0_0[1], ep00)
    e01_0 = padi(adj_0_1[0], ep01)
    e01_1 = padi(adj_0_1[1], ep01)
    e11_0 = padi(adj_1_1[0], ep11)
    e11_1 = padi(adj_1_1[1], ep11)

    z16 = jnp.zeros((n1 // _NS, 16), _F32)
    z64 = jnp.zeros((n1 // _NS, 64), _F32)
    ones2 = jnp.concatenate([jnp.ones((_CH, 16), _F32),
                             jnp.zeros((_CH, 16), _F32)], axis=0)

    (r00, s00, r01, s01, r11, s11, d00p, d01p, d11p) = _sc_prep(
        t0, t1, e00_0, e00_1, e01_0, e01_1, e11_0, e11_1,
        z16, ones2, n0, n1, (e00, e01, e11))

    stats = {"0_0": _tc_stats(r00, s00, e00, 2048),
             "0_1": _tc_stats(r01, s01, e01, 2048),
             "1_1": _tc_stats(r11, s11, e11, 2048)}
    scales, shifts = {}, {}
    for a, ecnt in (("0_0", e00), ("0_1", e01), ("1_1", e11)):
        sm = stats[a][0, :3] / ecnt
        sq = stats[a][0, 3:6] / ecnt
        var = sq - sm * sm
        g, bt = params["bn"][a]
        sc = g / jnp.sqrt(var + 1e-5)
        scales[a] = sc
        shifts[a] = bt - sm * sc

    layers = params["layers"]
    minv, cinv = {}, {}
    for a in ("0_0", "0_1", "1_1"):
        ms, cs = [], []
        for layer in layers:
            w1, b1 = layer["msg_" + a][0]
            w1i = w1[2 * hid:]
            ms.append(scales[a][:, None] * w1i)
            cs.append(shifts[a] @ w1i + b1)
        minv[a] = jnp.concatenate(ms, axis=0)
        cinv[a] = jnp.stack(cs, axis=0)

    ip00 = _tc_invp(r00, s00, minv["0_0"], cinv["0_0"], e00, 2048)
    ip01 = _tc_invp(r01, s01, minv["0_1"], cinv["0_1"], e01, 2048)
    ip11 = _tc_invp(r11, s11, minv["1_1"], cinv["1_1"], e11, 2048)

    d00 = d00p[0, :, 0] + d00p[1, :, 0]
    d01 = d01p[0, :, 0] + d01p[1, :, 0]
    d11 = d11p[0, :, 0] + d11p[1, :, 0]
    deg0 = jnp.concatenate([d00[:, None], d01[:, None],
                            jnp.ones((n0, 1), _F32),
                            jnp.zeros((n0, 125), _F32)], axis=1)
    deg1 = jnp.concatenate([d11[:, None], jnp.ones((n1, 1), _F32),
                            jnp.zeros((n1, 126), _F32)], axis=1)

    x0 = _tc_dense(x_0, params["emb_0"][0], params["emb_0"][1], 400)
    x1 = _tc_dense(x_1, params["emb_1"][0], params["emb_1"][1], 800)

    for li, layer in enumerate(layers):
        w00 = layer["msg_0_0"][0][0]
        w01 = layer["msg_0_1"][0][0]
        w11 = layer["msg_1_1"][0][0]
        wcat0 = jnp.concatenate([w00[:hid], w00[hid:2 * hid], w01[:hid]], axis=1)
        wcat1 = jnp.concatenate([w01[hid:2 * hid], w11[:hid], w11[hid:2 * hid]],
                                axis=1)
        p0a, p0b, p0c = _tc_proj(x0, wcat0, 400)
        p1a, p1b, p1c = _tc_proj(x1, wcat1, 800)
        r00t, s00t, r01t = (p.reshape(2 * n0, 64) for p in (p0a, p0b, p0c))
        s01t, r11t, s11t = (p.reshape(2 * n1, 64) for p in (p1a, p1b, p1c))

        a00, a01, a11 = _sc_edges(r00t, s00t, r01t, s01t, r11t, s11t,
                                  e00_0, e00_1, e01_0, e01_1, e11_0, e11_1,
                                  ip00, ip01, ip11, z64, li, n0, n1)

        (u1, ub1), (u2, ub2) = layer["upd_0"]
        w2_00, b2_00 = layer["msg_0_0"][1]
        w2_01, b2_01 = layer["msg_0_1"][1]
        u1x, u1a, u1b = u1[:hid], u1[hid:2 * hid], u1[2 * hid:]
        a00m = w2_00 @ u1a
        a01m = w2_01 @ u1b
        vmat0 = jnp.concatenate([(b2_00 @ u1a)[None], (b2_01 @ u1b)[None],
                                 ub1[None], jnp.zeros((125, 128), _F32)],
                                axis=0)
        x0 = _tc_update2(x0, a00, a01, deg0, u1x, a00m, a01m, vmat0, u2, ub2,
                         400)

        (v1, vb1), (v2, vb2) = layer["upd_1"]
        w2_11, b2_11 = layer["msg_1_1"][1]
        v1x, v1c = v1[:hid], v1[hid:]
        a11m = w2_11 @ v1c
        vmat1 = jnp.concatenate([(b2_11 @ v1c)[None], vb1[None],
                                 jnp.zeros((126, 128), _F32)], axis=0)
        x1 = _tc_update1(x1, a11, deg1, v1x, a11m, vmat1, v2, vb2, 800)

    y0 = _tc_mlp2(x0, params["pre_0"][0][0], params["pre_0"][0][1],
                  params["pre_0"][1][0], params["pre_0"][1][1], 400)
    y1 = _tc_mlp2(x1, params["pre_1"][0][0], params["pre_1"][0][1],
                  params["pre_1"][1][0], params["pre_1"][1][1], 800)

    ids0 = cell_0_batch.astype(jnp.int32).reshape(n0 // 400, 1, 400)
    ids1 = cell_1_batch.astype(jnp.int32).reshape(n1 // 800, 1, 800)
    p0 = _tc_pool(y0, ids0, 400)
    p1 = _tc_pool(y1, ids1, 800)

    out = _tc_post(p0, p1, params["post"][0][0], params["post"][0][1],
                   params["post"][1][0], params["post"][1][1])
    return jnp.squeeze(out)
